# baseline (speedup 1.0000x reference)
import functools

import jax
import jax.numpy as jnp
from jax import lax
from jax.experimental import pallas as pl
from jax.experimental.pallas import tpu as pltpu
from jax.experimental.pallas import tpu_sc as plsc

NC = 2
NS = 16
NW = NC * NS
LANES = 16
SEQ_PER_BLK = 2
SLOTS = 4


def _build(B, L, E):
    FLAT = B * L
    assert FLAT % NW == 0
    rows_per_w = FLAT // NW
    BLK = SEQ_PER_BLK * L
    assert rows_per_w % BLK == 0
    nblocks = rows_per_w // BLK
    assert nblocks % SLOTS == 0
    nsteps = nblocks // SLOTS
    CH = 100 if BLK % 100 == 0 else 128
    assert BLK % CH == 0
    NCH = BLK // CH
    chunks_per_w = rows_per_w // CH

    mesh = plsc.VectorSubcoreMesh(
        core_axis_name="c", subcore_axis_name="s",
        num_cores=NC, num_subcores=NS)

    @functools.partial(
        pl.kernel,
        out_type=jax.ShapeDtypeStruct((FLAT, E), jnp.float32),
        mesh=mesh,
        compiler_params=pltpu.CompilerParams(use_tc_tiling_on_sc=False),
        scratch_types=[
            pltpu.VMEM((SLOTS, NCH, CH), jnp.int32),
            pltpu.VMEM((SLOTS, BLK, E), jnp.float32),
            pltpu.VMEM((L, E), jnp.float32),
            pltpu.SemaphoreType.DMA((SLOTS,)),
            pltpu.SemaphoreType.DMA((SLOTS,)),
            pltpu.SemaphoreType.DMA((SLOTS,)),
        ],
    )
    def emb_kernel(idx_hbm, tok_hbm, pos_hbm, out_hbm,
                   idx_v, rows_v, pos_v, isems, gsems, osems):
        wid = lax.axis_index("s") * NC + lax.axis_index("c")
        chunk_base = wid * chunks_per_w
        row_base = wid * rows_per_w

        pltpu.sync_copy(pos_hbm, pos_v)

        def idx_issue(j, t):
            pltpu.async_copy(idx_hbm.at[pl.ds(chunk_base + j * NCH, NCH)],
                             idx_v.at[t], isems.at[t])

        def idx_wait(t):
            pltpu.make_async_copy(idx_hbm.at[pl.ds(0, NCH)],
                                  idx_v.at[t], isems.at[t]).wait()

        def gather_issue(s):
            for c in range(NCH):
                pltpu.async_copy(tok_hbm.at[idx_v.at[s, c]],
                                 rows_v.at[s, pl.ds(c * CH, CH)],
                                 gsems.at[s])

        def gather_wait(s):
            pltpu.make_async_copy(tok_hbm.at[pl.ds(0, BLK)],
                                  rows_v.at[s], gsems.at[s]).wait()

        def scatter_issue(j, s):
            pltpu.async_copy(rows_v.at[s],
                             out_hbm.at[pl.ds(row_base + j * BLK, BLK)],
                             osems.at[s])

        def scatter_wait(s):
            pltpu.make_async_copy(rows_v.at[s],
                                  out_hbm.at[pl.ds(0, BLK)],
                                  osems.at[s]).wait()

        def add_pos(s):
            def body(l, c):
                for k in range(E // LANES):
                    pv = pos_v[l, pl.ds(k * LANES, LANES)]
                    for q in range(SEQ_PER_BLK):
                        r = q * L + l
                        rows_v[s, r, pl.ds(k * LANES, LANES)] = (
                            rows_v[s, r, pl.ds(k * LANES, LANES)] + pv)
                return c
            lax.fori_loop(0, L, body, 0)

        pltpu.sync_copy(idx_hbm.at[pl.ds(chunk_base, NCH)], idx_v.at[0])
        gather_issue(0)
        idx_issue(1, 1)

        def step_body(g, carry):
            for s in range(SLOTS):
                j = SLOTS * g + s

                def do_step():
                    idx_wait(s)
                    gather_issue(s)

                def do_osem_wait():
                    scatter_wait(s)

                def do_idx_issue():
                    idx_issue(j + 1, (s + 1) % SLOTS)

                def do_compute():
                    sp = (s - 1) % SLOTS
                    gather_wait(sp)
                    add_pos(sp)
                    scatter_issue(j - 1, sp)

                if s == 0:
                    pl.when(g > 0)(do_osem_wait)
                    pl.when(g > 0)(do_step)
                    pl.when(g > 0)(do_idx_issue)
                    pl.when(g > 0)(do_compute)
                else:
                    pl.when(g > 0)(do_osem_wait)
                    do_step()
                    if s == SLOTS - 1:
                        pl.when(g < nsteps - 1)(do_idx_issue)
                    else:
                        do_idx_issue()
                    do_compute()
            return carry

        lax.fori_loop(0, nsteps, step_body, 0)

        last = SLOTS - 1
        gather_wait(last)
        add_pos(last)
        scatter_issue(nblocks - 1, last)
        for s in range(SLOTS):
            scatter_wait(s)

    return emb_kernel


@jax.jit
def kernel(inputs, token_table, pos_table):
    B, L = inputs.shape
    E = token_table.shape[1]
    CH = 100 if (SEQ_PER_BLK * L) % 100 == 0 else 128
    emb = _build(B, L, E)
    idx = inputs.reshape(B * L // CH, CH).astype(jnp.int32)
    out = emb(idx, token_table, pos_table)
    return out.reshape(B, L, E)
